# Initial kernel scaffold; baseline (speedup 1.0000x reference)
import jax
import jax.numpy as jnp
from jax.experimental import pallas as pl


def kernel(x, edge_index, BU_edge_index, rootindex, W1_td, W2_td, W1_bu, W2_bu):
    raise NotImplementedError("write your pallas kernel here")


# speedup vs baseline: 24.2547x; 24.2547x over previous
import functools

import jax
import jax.numpy as jnp
from jax import lax
from jax.experimental import pallas as pl
from jax.experimental.pallas import tpu as pltpu
from jax.experimental.pallas import tpu_sc as plsc

N = 10000
E = 320000
D_IN = 128
D_H = 64
D_OUT = 64

NC, NS = 2, 16
N_PAD = 10240
CHUNK = 80
EPT = E // NS
NCHUNK = EPT // CHUNK
ROWS_T = N_PAD // NS
ZROWS = 128

_sc_mesh = plsc.VectorSubcoreMesh(
    core_axis_name="c", subcore_axis_name="s", num_cores=NC, num_subcores=NS)


def _fill_1d(ref, val, n):
  def body(i, _):
    ref[pl.ds(i * 16, 16)] = jnp.full((16,), val, jnp.float32)
    return 0
  lax.fori_loop(0, n // 16, body, 0, unroll=False)


def _fill_2d(ref, val, rows, cols):
  def body(i, _):
    r = i // (cols // 16)
    k = i % (cols // 16)
    ref[r, pl.ds(k * 16, 16)] = jnp.full((16,), val, jnp.float32)
    return 0
  lax.fori_loop(0, rows * (cols // 16), body, 0, unroll=False)


@functools.partial(
    pl.kernel,
    out_type=(jax.ShapeDtypeStruct((N_PAD,), jnp.float32),
              jax.ShapeDtypeStruct((N_PAD,), jnp.float32)),
    mesh=_sc_mesh,
    scratch_types=(
        pltpu.VMEM((NCHUNK, CHUNK), jnp.int32),
        pltpu.VMEM((CHUNK,), jnp.float32),
        pltpu.VMEM((ROWS_T,), jnp.float32),
        pltpu.VMEM_SHARED((N_PAD,), jnp.float32),
    ),
    compiler_params=pltpu.CompilerParams(use_tc_tiling_on_sc=False),
)
def _sc_deg(dst_td, dst_bu, deg_td, deg_bu, idx_v, ones_v, zero1_v, sdeg):
  c = lax.axis_index("c")
  s = lax.axis_index("s")
  _fill_1d(ones_v, 1.0, CHUNK)
  _fill_1d(zero1_v, 0.0, ROWS_T)
  pltpu.sync_copy(zero1_v, sdeg.at[pl.ds(s * ROWS_T, ROWS_T)])

  @pl.when(c == 0)
  def _():
    pltpu.sync_copy(dst_td.at[s], idx_v)

  @pl.when(c == 1)
  def _():
    pltpu.sync_copy(dst_bu.at[s], idx_v)

  plsc.subcore_barrier()

  def body(j, _):
    pltpu.sync_copy(ones_v, sdeg.at[idx_v.at[j]], add=True)
    return 0
  lax.fori_loop(0, NCHUNK, body, 0, unroll=False)
  plsc.subcore_barrier()

  @pl.when(c == 0)
  def _():
    pltpu.sync_copy(sdeg.at[pl.ds(s * ROWS_T, ROWS_T)],
                    deg_td.at[pl.ds(s * ROWS_T, ROWS_T)])

  @pl.when(c == 1)
  def _():
    pltpu.sync_copy(sdeg.at[pl.ds(s * ROWS_T, ROWS_T)],
                    deg_bu.at[pl.ds(s * ROWS_T, ROWS_T)])


def _make_edge_pass(with_sd):
  out_type = [jax.ShapeDtypeStruct((N_PAD, D_H), jnp.float32),
              jax.ShapeDtypeStruct((N_PAD, D_H), jnp.float32)]
  scratch = [
      pltpu.VMEM((NCHUNK, CHUNK), jnp.int32),
      pltpu.VMEM((NCHUNK, CHUNK), jnp.int32),
      pltpu.VMEM((CHUNK, D_H), jnp.float32),
      pltpu.VMEM((ZROWS, D_H), jnp.float32),
      pltpu.VMEM_SHARED((N_PAD, D_H), jnp.float32),
  ]
  if with_sd:
    out_type += [jax.ShapeDtypeStruct((N_PAD,), jnp.float32),
                 jax.ShapeDtypeStruct((N_PAD,), jnp.float32)]
    scratch += [
        pltpu.VMEM((CHUNK,), jnp.float32),
        pltpu.VMEM((ROWS_T,), jnp.float32),
        pltpu.VMEM_SHARED((N_PAD,), jnp.float32),
        pltpu.VMEM_SHARED((N_PAD,), jnp.float32),
    ]

  def body(z_td, z_bu, src_td, dst_td, src_bu, dst_bu, *rest):
    if with_sd:
      (dinv_td, dinv_bu, s_td_out, s_bu_out, sd_td_out, sd_bu_out,
       src_v, dst_v, rows_v, zero2_v, sacc,
       val_v, zero1_v, sdinv, ssd) = rest
    else:
      (s_td_out, s_bu_out,
       src_v, dst_v, rows_v, zero2_v, sacc) = rest
    c = lax.axis_index("c")
    s = lax.axis_index("s")
    _fill_2d(zero2_v, 0.0, ZROWS, D_H)
    if with_sd:
      _fill_1d(zero1_v, 0.0, ROWS_T)

    def run(z_hbm, src_hbm, dst_hbm, s_out, dinv_hbm=None, sd_out=None):
      for k in range(ROWS_T // ZROWS):
        pltpu.sync_copy(zero2_v, sacc.at[pl.ds(s * ROWS_T + k * ZROWS, ZROWS)])
      if with_sd:
        pltpu.sync_copy(dinv_hbm.at[pl.ds(s * ROWS_T, ROWS_T)],
                        sdinv.at[pl.ds(s * ROWS_T, ROWS_T)])
        pltpu.sync_copy(zero1_v, ssd.at[pl.ds(s * ROWS_T, ROWS_T)])
      pltpu.sync_copy(src_hbm.at[s], src_v)
      pltpu.sync_copy(dst_hbm.at[s], dst_v)
      plsc.subcore_barrier()

      def chunk(j, _):
        pltpu.sync_copy(z_hbm.at[src_v.at[j]], rows_v)
        pltpu.sync_copy(rows_v, sacc.at[dst_v.at[j]], add=True)
        if with_sd:
          pltpu.sync_copy(sdinv.at[src_v.at[j]], val_v)
          pltpu.sync_copy(val_v, ssd.at[dst_v.at[j]], add=True)
        return 0
      lax.fori_loop(0, NCHUNK, chunk, 0, unroll=False)
      plsc.subcore_barrier()
      pltpu.sync_copy(sacc.at[pl.ds(s * ROWS_T, ROWS_T)],
                      s_out.at[pl.ds(s * ROWS_T, ROWS_T)])
      if with_sd:
        pltpu.sync_copy(ssd.at[pl.ds(s * ROWS_T, ROWS_T)],
                        sd_out.at[pl.ds(s * ROWS_T, ROWS_T)])

    @pl.when(c == 0)
    def _():
      if with_sd:
        run(z_td, src_td, dst_td, s_td_out, dinv_td, sd_td_out)
      else:
        run(z_td, src_td, dst_td, s_td_out)

    @pl.when(c == 1)
    def _():
      if with_sd:
        run(z_bu, src_bu, dst_bu, s_bu_out, dinv_bu, sd_bu_out)
      else:
        run(z_bu, src_bu, dst_bu, s_bu_out)

  return pl.kernel(
      body, out_type=tuple(out_type), mesh=_sc_mesh,
      scratch_types=tuple(scratch),
      compiler_params=pltpu.CompilerParams(use_tc_tiling_on_sc=False))


_sc_pass1 = _make_edge_pass(with_sd=True)
_sc_pass2 = _make_edge_pass(with_sd=False)


RB = 2048


def _tc_pre_body(x_ref, degtd_ref, degbu_ref, w1td_ref, w1bu_ref,
                 z1td_ref, z1bu_ref, dinvtd_ref, dinvbu_ref):
  x = x_ref[...]
  dtd = lax.rsqrt(degtd_ref[...] + 1.0)
  dbu = lax.rsqrt(degbu_ref[...] + 1.0)
  z1td_ref[...] = jnp.dot(x, w1td_ref[...],
                          preferred_element_type=jnp.float32) * dtd
  z1bu_ref[...] = jnp.dot(x, w1bu_ref[...],
                          preferred_element_type=jnp.float32) * dbu
  dinvtd_ref[...] = dtd
  dinvbu_ref[...] = dbu


_tc_pre = pl.pallas_call(
    _tc_pre_body,
    grid=(N_PAD // RB,),
    in_specs=[
        pl.BlockSpec((RB, D_IN), lambda i: (i, 0)),
        pl.BlockSpec((RB, 1), lambda i: (i, 0)),
        pl.BlockSpec((RB, 1), lambda i: (i, 0)),
        pl.BlockSpec((D_IN, D_H), lambda i: (0, 0)),
        pl.BlockSpec((D_IN, D_H), lambda i: (0, 0)),
    ],
    out_specs=[
        pl.BlockSpec((RB, D_H), lambda i: (i, 0)),
        pl.BlockSpec((RB, D_H), lambda i: (i, 0)),
        pl.BlockSpec((RB, 1), lambda i: (i, 0)),
        pl.BlockSpec((RB, 1), lambda i: (i, 0)),
    ],
    out_shape=[
        jax.ShapeDtypeStruct((N_PAD, D_H), jnp.float32),
        jax.ShapeDtypeStruct((N_PAD, D_H), jnp.float32),
        jax.ShapeDtypeStruct((N_PAD, 1), jnp.float32),
        jax.ShapeDtypeStruct((N_PAD, 1), jnp.float32),
    ],
)


def _tc_mid_body(s1td_ref, s1bu_ref, z1td_ref, z1bu_ref, dtd_ref, dbu_ref,
                 sdtd_ref, sdbu_ref, w2atd_ref, w2abu_ref, rootx_ref,
                 w2btd_ref, w2bbu_ref,
                 z2td_ref, z2bu_ref, ctd_ref, cbu_ref):
  def one(s1, z1, dinv, sd, w2a, w2b):
    h1 = jnp.maximum(dinv * (s1 + z1), 0.0)
    z2 = jnp.dot(h1, w2a, preferred_element_type=jnp.float32) * dinv
    rw = jnp.dot(rootx_ref[...], w2b, preferred_element_type=jnp.float32)
    cc = dinv * z2 + (dinv * (sd + dinv)) * rw
    return z2, cc

  z2td, ctd = one(s1td_ref[...], z1td_ref[...], dtd_ref[...], sdtd_ref[...],
                  w2atd_ref[...], w2btd_ref[...])
  z2bu, cbu = one(s1bu_ref[...], z1bu_ref[...], dbu_ref[...], sdbu_ref[...],
                  w2abu_ref[...], w2bbu_ref[...])
  z2td_ref[...] = z2td
  z2bu_ref[...] = z2bu
  ctd_ref[...] = ctd
  cbu_ref[...] = cbu


_tc_mid = pl.pallas_call(
    _tc_mid_body,
    grid=(N_PAD // RB,),
    in_specs=[
        pl.BlockSpec((RB, D_H), lambda i: (i, 0)),
        pl.BlockSpec((RB, D_H), lambda i: (i, 0)),
        pl.BlockSpec((RB, D_H), lambda i: (i, 0)),
        pl.BlockSpec((RB, D_H), lambda i: (i, 0)),
        pl.BlockSpec((RB, 1), lambda i: (i, 0)),
        pl.BlockSpec((RB, 1), lambda i: (i, 0)),
        pl.BlockSpec((RB, 1), lambda i: (i, 0)),
        pl.BlockSpec((RB, 1), lambda i: (i, 0)),
        pl.BlockSpec((D_H, D_OUT), lambda i: (0, 0)),
        pl.BlockSpec((D_H, D_OUT), lambda i: (0, 0)),
        pl.BlockSpec((1, D_IN), lambda i: (0, 0)),
        pl.BlockSpec((D_IN, D_OUT), lambda i: (0, 0)),
        pl.BlockSpec((D_IN, D_OUT), lambda i: (0, 0)),
    ],
    out_specs=[
        pl.BlockSpec((RB, D_OUT), lambda i: (i, 0)),
        pl.BlockSpec((RB, D_OUT), lambda i: (i, 0)),
        pl.BlockSpec((RB, D_OUT), lambda i: (i, 0)),
        pl.BlockSpec((RB, D_OUT), lambda i: (i, 0)),
    ],
    out_shape=[
        jax.ShapeDtypeStruct((N_PAD, D_OUT), jnp.float32),
        jax.ShapeDtypeStruct((N_PAD, D_OUT), jnp.float32),
        jax.ShapeDtypeStruct((N_PAD, D_OUT), jnp.float32),
        jax.ShapeDtypeStruct((N_PAD, D_OUT), jnp.float32),
    ],
)


def _tc_post_body(s2td_ref, s2bu_ref, ctd_ref, cbu_ref, dtd_ref, dbu_ref,
                  out_ref):
  td = jnp.maximum(dtd_ref[...] * s2td_ref[...] + ctd_ref[...], 0.0)
  bu = jnp.maximum(dbu_ref[...] * s2bu_ref[...] + cbu_ref[...], 0.0)
  out_ref[...] = jnp.concatenate([td, bu], axis=1)


_tc_post = pl.pallas_call(
    _tc_post_body,
    grid=(N_PAD // RB,),
    in_specs=[
        pl.BlockSpec((RB, D_OUT), lambda i: (i, 0)),
        pl.BlockSpec((RB, D_OUT), lambda i: (i, 0)),
        pl.BlockSpec((RB, D_OUT), lambda i: (i, 0)),
        pl.BlockSpec((RB, D_OUT), lambda i: (i, 0)),
        pl.BlockSpec((RB, 1), lambda i: (i, 0)),
        pl.BlockSpec((RB, 1), lambda i: (i, 0)),
    ],
    out_specs=pl.BlockSpec((RB, 2 * D_OUT), lambda i: (i, 0)),
    out_shape=jax.ShapeDtypeStruct((N_PAD, 2 * D_OUT), jnp.float32),
)


def kernel(x, edge_index, BU_edge_index, rootindex, W1_td, W2_td, W1_bu, W2_bu):
  src_td = edge_index[0].reshape(NS, NCHUNK, CHUNK)
  dst_td = edge_index[1].reshape(NS, NCHUNK, CHUNK)
  src_bu = BU_edge_index[0].reshape(NS, NCHUNK, CHUNK)
  dst_bu = BU_edge_index[1].reshape(NS, NCHUNK, CHUNK)
  x_p = jnp.pad(x, ((0, N_PAD - N), (0, 0)))

  deg_td, deg_bu = _sc_deg(dst_td, dst_bu)

  z1_td, z1_bu, dinv_td, dinv_bu = _tc_pre(
      x_p, deg_td[:, None], deg_bu[:, None], W1_td, W1_bu)

  s1_td, s1_bu, sd_td, sd_bu = _sc_pass1(
      z1_td, z1_bu, src_td, dst_td, src_bu, dst_bu,
      dinv_td.reshape(N_PAD), dinv_bu.reshape(N_PAD))

  root_x = lax.dynamic_slice_in_dim(x, rootindex[0], 1, axis=0)
  z2_td, z2_bu, c_td, c_bu = _tc_mid(
      s1_td, s1_bu, z1_td, z1_bu, dinv_td, dinv_bu,
      sd_td[:, None], sd_bu[:, None],
      W2_td[:D_H], W2_bu[:D_H], root_x, W2_td[D_H:], W2_bu[D_H:])

  s2_td, s2_bu = _sc_pass2(z2_td, z2_bu, src_td, dst_td, src_bu, dst_bu)

  out = _tc_post(s2_td, s2_bu, c_td, c_bu, dinv_td, dinv_bu)
  return out[:N]


# speedup vs baseline: 46.5096x; 1.9175x over previous
import functools

import jax
import jax.numpy as jnp
from jax import lax
from jax.experimental import pallas as pl
from jax.experimental.pallas import tpu as pltpu
from jax.experimental.pallas import tpu_sc as plsc

N = 10000
E = 320000
D_IN = 128
D_H = 64
D_OUT = 64
D_AUG = 80

NC, NS = 2, 16
N_PAD = 10240
CHUNK = 128
NCHUNK = 160
EPT = NCHUNK * CHUNK
E_PAD = NS * EPT
ROWS_T = N_PAD // NS
ZROWS = 128

_sc_mesh = plsc.VectorSubcoreMesh(
    core_axis_name="c", subcore_axis_name="s", num_cores=NC, num_subcores=NS)
_sc_params = pltpu.CompilerParams(use_tc_tiling_on_sc=False)


def _fill_1d(ref, val, n):
  def body(i, _):
    ref[pl.ds(i * 16, 16)] = jnp.full((16,), val, jnp.float32)
    return 0
  lax.fori_loop(0, n // 16, body, 0, unroll=False)


def _fill_2d(ref, val, rows, cols):
  def body(i, _):
    r = i // (cols // 16)
    k = i % (cols // 16)
    ref[r, pl.ds(k * 16, 16)] = jnp.full((16,), val, jnp.float32)
    return 0
  lax.fori_loop(0, rows * (cols // 16), body, 0, unroll=False)


@functools.partial(
    pl.kernel,
    out_type=(jax.ShapeDtypeStruct((N_PAD,), jnp.float32),
              jax.ShapeDtypeStruct((N_PAD,), jnp.float32)),
    mesh=_sc_mesh,
    scratch_types=(
        pltpu.VMEM((NCHUNK, CHUNK), jnp.int32),
        pltpu.VMEM((CHUNK,), jnp.float32),
        pltpu.VMEM((ROWS_T,), jnp.float32),
        pltpu.VMEM_SHARED((N_PAD,), jnp.float32),
        pltpu.SemaphoreType.DMA,
    ),
    compiler_params=_sc_params,
)
def _sc_deg(dst_td, dst_bu, deg_td, deg_bu, idx_v, ones_v, zero1_v, sdeg, sem):
  c = lax.axis_index("c")
  s = lax.axis_index("s")
  _fill_1d(ones_v, 1.0, CHUNK)
  _fill_1d(zero1_v, 0.0, ROWS_T)
  pltpu.sync_copy(zero1_v, sdeg.at[pl.ds(s * ROWS_T, ROWS_T)])

  @pl.when(c == 0)
  def _():
    pltpu.sync_copy(dst_td.at[s], idx_v)

  @pl.when(c == 1)
  def _():
    pltpu.sync_copy(dst_bu.at[s], idx_v)

  plsc.subcore_barrier()

  def block(b, _):
    for k in range(8):
      pltpu.async_copy(ones_v, sdeg.at[idx_v.at[b * 8 + k]], sem, add=True)
    for k in range(8):
      pltpu.make_async_copy(ones_v, sdeg.at[idx_v.at[b * 8 + k]], sem).wait()
    return 0
  lax.fori_loop(0, NCHUNK // 8, block, 0, unroll=False)
  plsc.subcore_barrier()

  @pl.when(c == 0)
  def _():
    pltpu.sync_copy(sdeg.at[pl.ds(s * ROWS_T, ROWS_T)],
                    deg_td.at[pl.ds(s * ROWS_T, ROWS_T)])

  @pl.when(c == 1)
  def _():
    pltpu.sync_copy(sdeg.at[pl.ds(s * ROWS_T, ROWS_T)],
                    deg_bu.at[pl.ds(s * ROWS_T, ROWS_T)])


def _make_edge_pass(width):
  out_type = (jax.ShapeDtypeStruct((N_PAD, width), jnp.float32),
              jax.ShapeDtypeStruct((N_PAD, width), jnp.float32))
  scratch = (
      pltpu.VMEM((NCHUNK, CHUNK), jnp.int32),
      pltpu.VMEM((NCHUNK, CHUNK), jnp.int32),
      pltpu.VMEM((CHUNK, width), jnp.float32),
      pltpu.VMEM((CHUNK, width), jnp.float32),
      pltpu.VMEM((ZROWS, width), jnp.float32),
      pltpu.VMEM_SHARED((N_PAD, width), jnp.float32),
      pltpu.SemaphoreType.DMA,
      pltpu.SemaphoreType.DMA,
  )

  def body(z_td, z_bu, src_td, dst_td, src_bu, dst_bu,
           out_td, out_bu, src_v, dst_v, buf0, buf1, zero2_v, sacc,
           sem0, sem1):
    c = lax.axis_index("c")
    s = lax.axis_index("s")
    _fill_2d(zero2_v, 0.0, ZROWS, width)

    def run(z_hbm, src_hbm, dst_hbm, s_out):
      for k in range(ROWS_T // ZROWS):
        pltpu.sync_copy(zero2_v, sacc.at[pl.ds(s * ROWS_T + k * ZROWS, ZROWS)])
      pltpu.sync_copy(src_hbm.at[s], src_v)
      pltpu.sync_copy(dst_hbm.at[s], dst_v)
      plsc.subcore_barrier()

      pltpu.async_copy(z_hbm.at[src_v.at[0]], buf0, sem0)

      def step(j, buf, sem, nbuf, nsem):
        @pl.when(j + 1 < NCHUNK)
        def _():
          pltpu.async_copy(z_hbm.at[src_v.at[j + 1]], nbuf, nsem)
        pltpu.make_async_copy(z_hbm.at[src_v.at[j]], buf, sem).wait()
        pltpu.sync_copy(buf, sacc.at[dst_v.at[j]], add=True)

      def pair(i, _):
        step(2 * i, buf0, sem0, buf1, sem1)
        step(2 * i + 1, buf1, sem1, buf0, sem0)
        return 0
      lax.fori_loop(0, NCHUNK // 2, pair, 0, unroll=False)
      plsc.subcore_barrier()
      pltpu.sync_copy(sacc.at[pl.ds(s * ROWS_T, ROWS_T)],
                      s_out.at[pl.ds(s * ROWS_T, ROWS_T)])

    @pl.when(c == 0)
    def _():
      run(z_td, src_td, dst_td, out_td)

    @pl.when(c == 1)
    def _():
      run(z_bu, src_bu, dst_bu, out_bu)

  return pl.kernel(body, out_type=out_type, mesh=_sc_mesh,
                   scratch_types=scratch, compiler_params=_sc_params)


_sc_pass1 = _make_edge_pass(D_AUG)
_sc_pass2 = _make_edge_pass(D_H)


RB = 2048


def _tc_pre_body(x_ref, degtd_ref, degbu_ref, w1td_ref, w1bu_ref,
                 z1td_ref, z1bu_ref, dinvtd_ref, dinvbu_ref):
  x = x_ref[...]
  dtd = lax.rsqrt(degtd_ref[...] + 1.0)
  dbu = lax.rsqrt(degbu_ref[...] + 1.0)
  zeros = jnp.zeros((x.shape[0], D_AUG - D_H - 1), jnp.float32)
  ztd = jnp.dot(x, w1td_ref[...], preferred_element_type=jnp.float32) * dtd
  zbu = jnp.dot(x, w1bu_ref[...], preferred_element_type=jnp.float32) * dbu
  z1td_ref[...] = jnp.concatenate([ztd, dtd, zeros], axis=1)
  z1bu_ref[...] = jnp.concatenate([zbu, dbu, zeros], axis=1)
  dinvtd_ref[...] = dtd
  dinvbu_ref[...] = dbu


_tc_pre = pl.pallas_call(
    _tc_pre_body,
    grid=(N_PAD // RB,),
    in_specs=[
        pl.BlockSpec((RB, D_IN), lambda i: (i, 0)),
        pl.BlockSpec((RB, 1), lambda i: (i, 0)),
        pl.BlockSpec((RB, 1), lambda i: (i, 0)),
        pl.BlockSpec((D_IN, D_H), lambda i: (0, 0)),
        pl.BlockSpec((D_IN, D_H), lambda i: (0, 0)),
    ],
    out_specs=[
        pl.BlockSpec((RB, D_AUG), lambda i: (i, 0)),
        pl.BlockSpec((RB, D_AUG), lambda i: (i, 0)),
        pl.BlockSpec((RB, 1), lambda i: (i, 0)),
        pl.BlockSpec((RB, 1), lambda i: (i, 0)),
    ],
    out_shape=[
        jax.ShapeDtypeStruct((N_PAD, D_AUG), jnp.float32),
        jax.ShapeDtypeStruct((N_PAD, D_AUG), jnp.float32),
        jax.ShapeDtypeStruct((N_PAD, 1), jnp.float32),
        jax.ShapeDtypeStruct((N_PAD, 1), jnp.float32),
    ],
)


def _tc_mid_body(s1td_ref, s1bu_ref, z1td_ref, z1bu_ref, dtd_ref, dbu_ref,
                 w2atd_ref, w2abu_ref, rootx_ref, w2btd_ref, w2bbu_ref,
                 z2td_ref, z2bu_ref, ctd_ref, cbu_ref):
  def one(s1aug, z1aug, dinv, w2a, w2b):
    s1 = s1aug[:, :D_H]
    sd = s1aug[:, D_H:D_H + 1]
    z1 = z1aug[:, :D_H]
    h1 = jnp.maximum(dinv * (s1 + z1), 0.0)
    z2 = jnp.dot(h1, w2a, preferred_element_type=jnp.float32) * dinv
    rw = jnp.dot(rootx_ref[...], w2b, preferred_element_type=jnp.float32)
    cc = dinv * z2 + (dinv * (sd + dinv)) * rw
    return z2, cc

  z2td, ctd = one(s1td_ref[...], z1td_ref[...], dtd_ref[...],
                  w2atd_ref[...], w2btd_ref[...])
  z2bu, cbu = one(s1bu_ref[...], z1bu_ref[...], dbu_ref[...],
                  w2abu_ref[...], w2bbu_ref[...])
  z2td_ref[...] = z2td
  z2bu_ref[...] = z2bu
  ctd_ref[...] = ctd
  cbu_ref[...] = cbu


_tc_mid = pl.pallas_call(
    _tc_mid_body,
    grid=(N_PAD // RB,),
    in_specs=[
        pl.BlockSpec((RB, D_AUG), lambda i: (i, 0)),
        pl.BlockSpec((RB, D_AUG), lambda i: (i, 0)),
        pl.BlockSpec((RB, D_AUG), lambda i: (i, 0)),
        pl.BlockSpec((RB, D_AUG), lambda i: (i, 0)),
        pl.BlockSpec((RB, 1), lambda i: (i, 0)),
        pl.BlockSpec((RB, 1), lambda i: (i, 0)),
        pl.BlockSpec((D_H, D_OUT), lambda i: (0, 0)),
        pl.BlockSpec((D_H, D_OUT), lambda i: (0, 0)),
        pl.BlockSpec((1, D_IN), lambda i: (0, 0)),
        pl.BlockSpec((D_IN, D_OUT), lambda i: (0, 0)),
        pl.BlockSpec((D_IN, D_OUT), lambda i: (0, 0)),
    ],
    out_specs=[
        pl.BlockSpec((RB, D_OUT), lambda i: (i, 0)),
        pl.BlockSpec((RB, D_OUT), lambda i: (i, 0)),
        pl.BlockSpec((RB, D_OUT), lambda i: (i, 0)),
        pl.BlockSpec((RB, D_OUT), lambda i: (i, 0)),
    ],
    out_shape=[
        jax.ShapeDtypeStruct((N_PAD, D_OUT), jnp.float32),
        jax.ShapeDtypeStruct((N_PAD, D_OUT), jnp.float32),
        jax.ShapeDtypeStruct((N_PAD, D_OUT), jnp.float32),
        jax.ShapeDtypeStruct((N_PAD, D_OUT), jnp.float32),
    ],
)


def _tc_post_body(s2td_ref, s2bu_ref, ctd_ref, cbu_ref, dtd_ref, dbu_ref,
                  out_ref):
  td = jnp.maximum(dtd_ref[...] * s2td_ref[...] + ctd_ref[...], 0.0)
  bu = jnp.maximum(dbu_ref[...] * s2bu_ref[...] + cbu_ref[...], 0.0)
  out_ref[...] = jnp.concatenate([td, bu], axis=1)


_tc_post = pl.pallas_call(
    _tc_post_body,
    grid=(N_PAD // RB,),
    in_specs=[
        pl.BlockSpec((RB, D_OUT), lambda i: (i, 0)),
        pl.BlockSpec((RB, D_OUT), lambda i: (i, 0)),
        pl.BlockSpec((RB, D_OUT), lambda i: (i, 0)),
        pl.BlockSpec((RB, D_OUT), lambda i: (i, 0)),
        pl.BlockSpec((RB, 1), lambda i: (i, 0)),
        pl.BlockSpec((RB, 1), lambda i: (i, 0)),
    ],
    out_specs=pl.BlockSpec((RB, 2 * D_OUT), lambda i: (i, 0)),
    out_shape=jax.ShapeDtypeStruct((N_PAD, 2 * D_OUT), jnp.float32),
)


def _pad_edges(ei):
  pad = N + (jnp.arange(E_PAD - E, dtype=jnp.int32) % (N_PAD - N))
  src = jnp.concatenate([ei[0], pad]).reshape(NS, NCHUNK, CHUNK)
  dst = jnp.concatenate([ei[1], pad]).reshape(NS, NCHUNK, CHUNK)
  return src, dst


def kernel(x, edge_index, BU_edge_index, rootindex, W1_td, W2_td, W1_bu, W2_bu):
  src_td, dst_td = _pad_edges(edge_index)
  src_bu, dst_bu = _pad_edges(BU_edge_index)
  x_p = jnp.pad(x, ((0, N_PAD - N), (0, 0)))

  deg_td, deg_bu = _sc_deg(dst_td, dst_bu)

  z1_td, z1_bu, dinv_td, dinv_bu = _tc_pre(
      x_p, deg_td[:, None], deg_bu[:, None], W1_td, W1_bu)

  s1_td, s1_bu = _sc_pass1(z1_td, z1_bu, src_td, dst_td, src_bu, dst_bu)

  root_x = lax.dynamic_slice_in_dim(x, rootindex[0], 1, axis=0)
  z2_td, z2_bu, c_td, c_bu = _tc_mid(
      s1_td, s1_bu, z1_td, z1_bu, dinv_td, dinv_bu,
      W2_td[:D_H], W2_bu[:D_H], root_x, W2_td[D_H:], W2_bu[D_H:])

  s2_td, s2_bu = _sc_pass2(z2_td, z2_bu, src_td, dst_td, src_bu, dst_bu)

  out = _tc_post(s2_td, s2_bu, c_td, c_bu, dinv_td, dinv_bu)
  return out[:N]


# speedup vs baseline: 52.4202x; 1.1271x over previous
import functools

import jax
import jax.numpy as jnp
from jax import lax
from jax.experimental import pallas as pl
from jax.experimental.pallas import tpu as pltpu
from jax.experimental.pallas import tpu_sc as plsc

N = 10000
E = 320000
D_IN = 128
D_H = 64
D_OUT = 64
D_AUG = 96

NC, NS = 2, 16
N_PAD = 10240
CHUNK = 128
NCHUNK = 160
EPT = NCHUNK * CHUNK
E_PAD = NS * EPT
ROWS_T = N_PAD // NS
ZROWS = 128

_sc_mesh = plsc.VectorSubcoreMesh(
    core_axis_name="c", subcore_axis_name="s", num_cores=NC, num_subcores=NS)
_sc_params = pltpu.CompilerParams(use_tc_tiling_on_sc=False)


def _fill_1d(ref, val, n):
  def body(i, _):
    ref[pl.ds(i * 16, 16)] = jnp.full((16,), val, jnp.float32)
    return 0
  lax.fori_loop(0, n // 16, body, 0, unroll=False)


def _fill_2d(ref, val, rows, cols):
  def body(i, _):
    r = i // (cols // 16)
    k = i % (cols // 16)
    ref[r, pl.ds(k * 16, 16)] = jnp.full((16,), val, jnp.float32)
    return 0
  lax.fori_loop(0, rows * (cols // 16), body, 0, unroll=False)


@functools.partial(
    pl.kernel,
    out_type=(jax.ShapeDtypeStruct((N_PAD,), jnp.float32),
              jax.ShapeDtypeStruct((N_PAD,), jnp.float32)),
    mesh=_sc_mesh,
    scratch_types=(
        pltpu.VMEM((NCHUNK, CHUNK), jnp.int32),
        pltpu.VMEM((CHUNK,), jnp.float32),
        pltpu.VMEM((ROWS_T,), jnp.float32),
        pltpu.VMEM_SHARED((N_PAD,), jnp.float32),
        pltpu.SemaphoreType.DMA,
    ),
    compiler_params=_sc_params,
)
def _sc_deg(dst_td, dst_bu, deg_td, deg_bu, idx_v, ones_v, zero1_v, sdeg, sem):
  c = lax.axis_index("c")
  s = lax.axis_index("s")
  _fill_1d(ones_v, 1.0, CHUNK)
  _fill_1d(zero1_v, 0.0, ROWS_T)
  pltpu.sync_copy(zero1_v, sdeg.at[pl.ds(s * ROWS_T, ROWS_T)])

  @pl.when(c == 0)
  def _():
    pltpu.sync_copy(dst_td.at[s], idx_v)

  @pl.when(c == 1)
  def _():
    pltpu.sync_copy(dst_bu.at[s], idx_v)

  plsc.subcore_barrier()

  def block(b, _):
    for k in range(8):
      pltpu.async_copy(ones_v, sdeg.at[idx_v.at[b * 8 + k]], sem, add=True)
    for k in range(8):
      pltpu.make_async_copy(ones_v, sdeg.at[idx_v.at[b * 8 + k]], sem).wait()
    return 0
  lax.fori_loop(0, NCHUNK // 8, block, 0, unroll=False)
  plsc.subcore_barrier()

  @pl.when(c == 0)
  def _():
    pltpu.sync_copy(sdeg.at[pl.ds(s * ROWS_T, ROWS_T)],
                    deg_td.at[pl.ds(s * ROWS_T, ROWS_T)])

  @pl.when(c == 1)
  def _():
    pltpu.sync_copy(sdeg.at[pl.ds(s * ROWS_T, ROWS_T)],
                    deg_bu.at[pl.ds(s * ROWS_T, ROWS_T)])


def _make_edge_pass(width):
  dt = jnp.bfloat16
  out_type = (jax.ShapeDtypeStruct((N_PAD, width), dt),
              jax.ShapeDtypeStruct((N_PAD, width), dt))
  scratch = (
      pltpu.VMEM((NCHUNK, CHUNK), jnp.int32),
      pltpu.VMEM((NCHUNK, CHUNK), jnp.int32),
      pltpu.VMEM((CHUNK, width), dt),
      pltpu.VMEM((CHUNK, width), dt),
      pltpu.VMEM_SHARED((N_PAD, width), dt),
      pltpu.SemaphoreType.DMA,
      pltpu.SemaphoreType.DMA,
  )

  def body(z_td, z_bu, src_td, dst_td, src_bu, dst_bu, zrows,
           out_td, out_bu, src_v, dst_v, buf0, buf1, sacc,
           sem0, sem1):
    c = lax.axis_index("c")
    s = lax.axis_index("s")

    def run(z_hbm, src_hbm, dst_hbm, s_out):
      pltpu.sync_copy(zrows, sacc.at[pl.ds(s * ROWS_T, ROWS_T)])
      pltpu.sync_copy(src_hbm.at[s], src_v)
      pltpu.sync_copy(dst_hbm.at[s], dst_v)
      plsc.subcore_barrier()

      pltpu.async_copy(z_hbm.at[src_v.at[0]], buf0, sem0)

      def step(j, buf, sem, nbuf, nsem):
        @pl.when(j + 1 < NCHUNK)
        def _():
          pltpu.async_copy(z_hbm.at[src_v.at[j + 1]], nbuf, nsem)
        pltpu.make_async_copy(z_hbm.at[src_v.at[j]], buf, sem).wait()
        pltpu.sync_copy(buf, sacc.at[dst_v.at[j]], add=True)

      def pair(i, _):
        step(2 * i, buf0, sem0, buf1, sem1)
        step(2 * i + 1, buf1, sem1, buf0, sem0)
        return 0
      lax.fori_loop(0, NCHUNK // 2, pair, 0, unroll=False)
      plsc.subcore_barrier()
      pltpu.sync_copy(sacc.at[pl.ds(s * ROWS_T, ROWS_T)],
                      s_out.at[pl.ds(s * ROWS_T, ROWS_T)])

    @pl.when(c == 0)
    def _():
      run(z_td, src_td, dst_td, out_td)

    @pl.when(c == 1)
    def _():
      run(z_bu, src_bu, dst_bu, out_bu)

  return pl.kernel(body, out_type=out_type, mesh=_sc_mesh,
                   scratch_types=scratch, compiler_params=_sc_params)


_sc_pass1 = _make_edge_pass(D_AUG)
_sc_pass2 = _make_edge_pass(D_H)


RB = 2048


def _tc_pre_body(x_ref, degtd_ref, degbu_ref, w1td_ref, w1bu_ref,
                 z1td_ref, z1bu_ref, dinvtd_ref, dinvbu_ref):
  x = x_ref[...]
  dtd = lax.rsqrt(degtd_ref[...] + 1.0)
  dbu = lax.rsqrt(degbu_ref[...] + 1.0)
  zeros = jnp.zeros((x.shape[0], D_AUG - D_H - 1), jnp.float32)
  ztd = jnp.dot(x, w1td_ref[...], preferred_element_type=jnp.float32) * dtd
  zbu = jnp.dot(x, w1bu_ref[...], preferred_element_type=jnp.float32) * dbu
  z1td_ref[...] = jnp.concatenate([ztd, dtd, zeros], axis=1).astype(jnp.bfloat16)
  z1bu_ref[...] = jnp.concatenate([zbu, dbu, zeros], axis=1).astype(jnp.bfloat16)
  dinvtd_ref[...] = dtd
  dinvbu_ref[...] = dbu


_tc_pre = pl.pallas_call(
    _tc_pre_body,
    grid=(N_PAD // RB,),
    in_specs=[
        pl.BlockSpec((RB, D_IN), lambda i: (i, 0)),
        pl.BlockSpec((RB, 1), lambda i: (i, 0)),
        pl.BlockSpec((RB, 1), lambda i: (i, 0)),
        pl.BlockSpec((D_IN, D_H), lambda i: (0, 0)),
        pl.BlockSpec((D_IN, D_H), lambda i: (0, 0)),
    ],
    out_specs=[
        pl.BlockSpec((RB, D_AUG), lambda i: (i, 0)),
        pl.BlockSpec((RB, D_AUG), lambda i: (i, 0)),
        pl.BlockSpec((RB, 1), lambda i: (i, 0)),
        pl.BlockSpec((RB, 1), lambda i: (i, 0)),
    ],
    out_shape=[
        jax.ShapeDtypeStruct((N_PAD, D_AUG), jnp.bfloat16),
        jax.ShapeDtypeStruct((N_PAD, D_AUG), jnp.bfloat16),
        jax.ShapeDtypeStruct((N_PAD, 1), jnp.float32),
        jax.ShapeDtypeStruct((N_PAD, 1), jnp.float32),
    ],
)


def _tc_mid_body(s1td_ref, s1bu_ref, z1td_ref, z1bu_ref, dtd_ref, dbu_ref,
                 w2atd_ref, w2abu_ref, rootx_ref, w2btd_ref, w2bbu_ref,
                 z2td_ref, z2bu_ref, ctd_ref, cbu_ref):
  def one(s1aug, z1aug, dinv, w2a, w2b):
    s1aug = s1aug.astype(jnp.float32)
    z1aug = z1aug.astype(jnp.float32)
    s1 = s1aug[:, :D_H]
    sd = s1aug[:, D_H:D_H + 1]
    z1 = z1aug[:, :D_H]
    h1 = jnp.maximum(dinv * (s1 + z1), 0.0)
    z2 = jnp.dot(h1, w2a, preferred_element_type=jnp.float32) * dinv
    rw = jnp.dot(rootx_ref[...], w2b, preferred_element_type=jnp.float32)
    cc = dinv * z2 + (dinv * (sd + dinv)) * rw
    return z2.astype(jnp.bfloat16), cc

  z2td, ctd = one(s1td_ref[...], z1td_ref[...], dtd_ref[...],
                  w2atd_ref[...], w2btd_ref[...])
  z2bu, cbu = one(s1bu_ref[...], z1bu_ref[...], dbu_ref[...],
                  w2abu_ref[...], w2bbu_ref[...])
  z2td_ref[...] = z2td
  z2bu_ref[...] = z2bu
  ctd_ref[...] = ctd
  cbu_ref[...] = cbu


_tc_mid = pl.pallas_call(
    _tc_mid_body,
    grid=(N_PAD // RB,),
    in_specs=[
        pl.BlockSpec((RB, D_AUG), lambda i: (i, 0)),
        pl.BlockSpec((RB, D_AUG), lambda i: (i, 0)),
        pl.BlockSpec((RB, D_AUG), lambda i: (i, 0)),
        pl.BlockSpec((RB, D_AUG), lambda i: (i, 0)),
        pl.BlockSpec((RB, 1), lambda i: (i, 0)),
        pl.BlockSpec((RB, 1), lambda i: (i, 0)),
        pl.BlockSpec((D_H, D_OUT), lambda i: (0, 0)),
        pl.BlockSpec((D_H, D_OUT), lambda i: (0, 0)),
        pl.BlockSpec((1, D_IN), lambda i: (0, 0)),
        pl.BlockSpec((D_IN, D_OUT), lambda i: (0, 0)),
        pl.BlockSpec((D_IN, D_OUT), lambda i: (0, 0)),
    ],
    out_specs=[
        pl.BlockSpec((RB, D_OUT), lambda i: (i, 0)),
        pl.BlockSpec((RB, D_OUT), lambda i: (i, 0)),
        pl.BlockSpec((RB, D_OUT), lambda i: (i, 0)),
        pl.BlockSpec((RB, D_OUT), lambda i: (i, 0)),
    ],
    out_shape=[
        jax.ShapeDtypeStruct((N_PAD, D_OUT), jnp.bfloat16),
        jax.ShapeDtypeStruct((N_PAD, D_OUT), jnp.bfloat16),
        jax.ShapeDtypeStruct((N_PAD, D_OUT), jnp.float32),
        jax.ShapeDtypeStruct((N_PAD, D_OUT), jnp.float32),
    ],
)


def _tc_post_body(s2td_ref, s2bu_ref, ctd_ref, cbu_ref, dtd_ref, dbu_ref,
                  out_ref):
  td = jnp.maximum(dtd_ref[...] * s2td_ref[...].astype(jnp.float32)
                   + ctd_ref[...], 0.0)
  bu = jnp.maximum(dbu_ref[...] * s2bu_ref[...].astype(jnp.float32)
                   + cbu_ref[...], 0.0)
  out_ref[...] = jnp.concatenate([td, bu], axis=1)


_tc_post = pl.pallas_call(
    _tc_post_body,
    grid=(N_PAD // RB,),
    in_specs=[
        pl.BlockSpec((RB, D_OUT), lambda i: (i, 0)),
        pl.BlockSpec((RB, D_OUT), lambda i: (i, 0)),
        pl.BlockSpec((RB, D_OUT), lambda i: (i, 0)),
        pl.BlockSpec((RB, D_OUT), lambda i: (i, 0)),
        pl.BlockSpec((RB, 1), lambda i: (i, 0)),
        pl.BlockSpec((RB, 1), lambda i: (i, 0)),
    ],
    out_specs=pl.BlockSpec((RB, 2 * D_OUT), lambda i: (i, 0)),
    out_shape=jax.ShapeDtypeStruct((N_PAD, 2 * D_OUT), jnp.float32),
)


def _pad_edges(ei):
  pad = N + (jnp.arange(E_PAD - E, dtype=jnp.int32) % (N_PAD - N))
  src = jnp.concatenate([ei[0], pad]).reshape(NS, NCHUNK, CHUNK)
  dst = jnp.concatenate([ei[1], pad]).reshape(NS, NCHUNK, CHUNK)
  return src, dst


def kernel(x, edge_index, BU_edge_index, rootindex, W1_td, W2_td, W1_bu, W2_bu):
  src_td, dst_td = _pad_edges(edge_index)
  src_bu, dst_bu = _pad_edges(BU_edge_index)
  x_p = jnp.pad(x, ((0, N_PAD - N), (0, 0)))

  deg_td, deg_bu = _sc_deg(dst_td, dst_bu)

  z1_td, z1_bu, dinv_td, dinv_bu = _tc_pre(
      x_p, deg_td[:, None], deg_bu[:, None], W1_td, W1_bu)

  z1rows = jnp.zeros((ROWS_T, D_AUG), jnp.bfloat16)
  z2rows = jnp.zeros((ROWS_T, D_H), jnp.bfloat16)
  s1_td, s1_bu = _sc_pass1(z1_td, z1_bu, src_td, dst_td, src_bu, dst_bu,
                           z1rows)

  root_x = lax.dynamic_slice_in_dim(x, rootindex[0], 1, axis=0)
  z2_td, z2_bu, c_td, c_bu = _tc_mid(
      s1_td, s1_bu, z1_td, z1_bu, dinv_td, dinv_bu,
      W2_td[:D_H], W2_bu[:D_H], root_x, W2_td[D_H:], W2_bu[D_H:])

  s2_td, s2_bu = _sc_pass2(z2_td, z2_bu, src_td, dst_td, src_bu, dst_bu,
                           z2rows)

  out = _tc_post(s2_td, s2_bu, c_td, c_bu, dinv_td, dinv_bu)
  return out[:N]
